# baseline (speedup 1.0000x reference)
import jax
import jax.numpy as jnp
from jax.experimental import pallas as pl

N = 1024
OUT_F = 64
ALPHA = 0.2


def _gat_body(x_ref, adj_ref, w_ref, ap_ref, bias_ref, wres_ref,
              gamma_ref, beta_ref, out_ref):
    x = x_ref[...]

    h = jnp.dot(x, w_ref[...], preferred_element_type=jnp.float32) \
        + bias_ref[...].reshape(1, OUT_F)

    f = jnp.sum(h * ap_ref[:, :OUT_F], axis=1, keepdims=True)
    g = jnp.sum(h * ap_ref[:, OUT_F:], axis=1, keepdims=True)
    t = jnp.max(f) + jnp.max(g)
    c = jnp.maximum(t, ALPHA * t)

    s = f + g.T
    s = jnp.maximum(s, ALPHA * s)
    e = jnp.where(adj_ref[...] != 0, jnp.exp(s - c), 0.0)
    corr = 1e-8 * jnp.max(e)

    haug = jnp.concatenate([h, jnp.ones((N, 1), jnp.float32)], axis=1)
    aaug = jnp.dot(e, haug, preferred_element_type=jnp.float32)

    res = jax.lax.dot_general(x, wres_ref[...],
                              (((1,), (1,)), ((), ())),
                              preferred_element_type=jnp.float32)
    hp = aaug[:, :OUT_F] / (aaug[:, OUT_F:] + corr) + res

    mean = jnp.mean(hp, axis=-1, keepdims=True)
    cen = hp - mean
    var = jnp.mean(cen * cen, axis=-1, keepdims=True)
    hn = cen * jax.lax.rsqrt(var + 1e-5) \
        * gamma_ref[...].reshape(1, OUT_F) \
        + beta_ref[...].reshape(1, OUT_F)

    out_ref[...] = jnp.where(hn > 0, hn, jnp.exp(jnp.minimum(hn, 0.0)) - 1.0)


def kernel(input, adj, W, a_param, bias, W_res, ln_gamma, ln_beta):
    return pl.pallas_call(
        _gat_body,
        out_shape=jax.ShapeDtypeStruct((N, OUT_F), jnp.float32),
    )(input, adj, W, a_param, bias, W_res, ln_gamma, ln_beta)


# speedup vs baseline: 1.0916x; 1.0916x over previous
import jax
import jax.numpy as jnp
from jax.experimental import pallas as pl
from jax.experimental.pallas import tpu as pltpu

N = 1024
OUT_F = 64
ALPHA = 0.2
BR = 256
K = N // BR


def _adj_copy(adj_ref, abuf, sem, k):
    return pltpu.make_async_copy(
        adj_ref.at[pl.ds(k * BR, BR), :], abuf.at[k % 2], sem.at[k % 2])


def _gat_body(x_ref, adj_ref, w_ref, ap_ref, bias_ref, wres_ref,
              gamma_ref, beta_ref, out_ref, abuf, sem):
    _adj_copy(adj_ref, abuf, sem, 0).start()

    x = x_ref[...]
    h = jnp.dot(x, w_ref[...], preferred_element_type=jnp.float32) \
        + bias_ref[...].reshape(1, OUT_F)
    f = jnp.sum(h * ap_ref[:, :OUT_F], axis=1, keepdims=True)
    g = jnp.sum(h * ap_ref[:, OUT_F:], axis=1, keepdims=True)
    t = jnp.max(f) + jnp.max(g)
    c = jnp.maximum(t, ALPHA * t)
    gt = g.T
    res = jax.lax.dot_general(x, wres_ref[...],
                              (((1,), (1,)), ((), ())),
                              preferred_element_type=jnp.float32)

    blocks = []
    rsums = []
    gmax = jnp.zeros((), jnp.float32)
    for k in range(K):
        if k + 1 < K:
            _adj_copy(adj_ref, abuf, sem, k + 1).start()
        _adj_copy(adj_ref, abuf, sem, k).wait()
        s = f[k * BR:(k + 1) * BR, :] + gt
        s = jnp.maximum(s, ALPHA * s)
        e = jnp.where(abuf[k % 2] != 0, jnp.exp(s - c), 0.0)
        gmax = jnp.maximum(gmax, jnp.max(e))
        rsums.append(jnp.sum(e, axis=1, keepdims=True))
        blocks.append(jnp.dot(e, h, preferred_element_type=jnp.float32))

    corr = 1e-8 * gmax
    hp = jnp.concatenate(blocks, axis=0) \
        / (jnp.concatenate(rsums, axis=0) + corr) + res

    mean = jnp.mean(hp, axis=-1, keepdims=True)
    cen = hp - mean
    var = jnp.mean(cen * cen, axis=-1, keepdims=True)
    hn = cen * jax.lax.rsqrt(var + 1e-5) \
        * gamma_ref[...].reshape(1, OUT_F) \
        + beta_ref[...].reshape(1, OUT_F)

    out_ref[...] = jnp.where(hn > 0, hn, jnp.exp(jnp.minimum(hn, 0.0)) - 1.0)


def kernel(input, adj, W, a_param, bias, W_res, ln_gamma, ln_beta):
    vmem = lambda: pl.BlockSpec(memory_space=pltpu.MemorySpace.HBM)
    return pl.pallas_call(
        _gat_body,
        in_specs=[
            pl.BlockSpec((N, 256), lambda: (0, 0)),
            vmem(),
            pl.BlockSpec((256, OUT_F), lambda: (0, 0)),
            pl.BlockSpec((1, 2 * OUT_F), lambda: (0, 0)),
            pl.BlockSpec((OUT_F,), lambda: (0,)),
            pl.BlockSpec((OUT_F, 256), lambda: (0, 0)),
            pl.BlockSpec((OUT_F,), lambda: (0,)),
            pl.BlockSpec((OUT_F,), lambda: (0,)),
        ],
        out_shape=jax.ShapeDtypeStruct((N, OUT_F), jnp.float32),
        scratch_shapes=[
            pltpu.VMEM((2, BR, N), jnp.int32),
            pltpu.SemaphoreType.DMA((2,)),
        ],
    )(input, adj, W, a_param, bias, W_res, ln_gamma, ln_beta)


# speedup vs baseline: 1.1714x; 1.0731x over previous
import jax
import jax.numpy as jnp
from jax.experimental import pallas as pl
from jax.experimental.pallas import tpu as pltpu

N = 1024
OUT_F = 64
ALPHA = 0.2
BR = 256
K = N // BR


def _adj_copy(adj_ref, abuf, sem, k):
    return pltpu.make_async_copy(
        adj_ref.at[pl.ds(k * BR, BR), :], abuf.at[k], sem.at[k])


def _gat_body(x_ref, adj_ref, w_ref, ap_ref, bias_ref, wres_ref,
              gamma_ref, beta_ref, out_ref, abuf, sem):
    for k in range(K):
        _adj_copy(adj_ref, abuf, sem, k).start()

    x = x_ref[...]
    h = jnp.dot(x, w_ref[...], preferred_element_type=jnp.float32) \
        + bias_ref[...].reshape(1, OUT_F)
    f = jnp.sum(h * ap_ref[:, :OUT_F], axis=1, keepdims=True)
    g = jnp.sum(h * ap_ref[:, OUT_F:], axis=1, keepdims=True)
    t = jnp.max(f) + jnp.max(g)
    c = jnp.maximum(t, ALPHA * t)
    gt = g.T
    res = jax.lax.dot_general(x, wres_ref[...],
                              (((1,), (1,)), ((), ())),
                              preferred_element_type=jnp.float32)

    blocks = []
    rsums = []
    gmax = jnp.zeros((), jnp.float32)
    for k in range(K):
        _adj_copy(adj_ref, abuf, sem, k).wait()
        s = f[k * BR:(k + 1) * BR, :] + gt
        s = jnp.maximum(s, ALPHA * s)
        e = jnp.where(abuf[k] != 0, jnp.exp(s - c), 0.0)
        gmax = jnp.maximum(gmax, jnp.max(e))
        rsums.append(jnp.sum(e, axis=1, keepdims=True))
        blocks.append(jnp.dot(e, h, preferred_element_type=jnp.float32))

    corr = 1e-8 * gmax
    hp = jnp.concatenate(blocks, axis=0) \
        / (jnp.concatenate(rsums, axis=0) + corr) + res

    mean = jnp.mean(hp, axis=-1, keepdims=True)
    cen = hp - mean
    var = jnp.mean(cen * cen, axis=-1, keepdims=True)
    hn = cen * jax.lax.rsqrt(var + 1e-5) \
        * gamma_ref[...].reshape(1, OUT_F) \
        + beta_ref[...].reshape(1, OUT_F)

    out_ref[...] = jnp.where(hn > 0, hn, jnp.exp(jnp.minimum(hn, 0.0)) - 1.0)


def kernel(input, adj, W, a_param, bias, W_res, ln_gamma, ln_beta):
    vmem = lambda: pl.BlockSpec(memory_space=pltpu.MemorySpace.HBM)
    return pl.pallas_call(
        _gat_body,
        in_specs=[
            pl.BlockSpec((N, 256), lambda: (0, 0)),
            vmem(),
            pl.BlockSpec((256, OUT_F), lambda: (0, 0)),
            pl.BlockSpec((1, 2 * OUT_F), lambda: (0, 0)),
            pl.BlockSpec((OUT_F,), lambda: (0,)),
            pl.BlockSpec((OUT_F, 256), lambda: (0, 0)),
            pl.BlockSpec((OUT_F,), lambda: (0,)),
            pl.BlockSpec((OUT_F,), lambda: (0,)),
        ],
        out_shape=jax.ShapeDtypeStruct((N, OUT_F), jnp.float32),
        scratch_shapes=[
            pltpu.VMEM((K, BR, N), jnp.int32),
            pltpu.SemaphoreType.DMA((K,)),
        ],
    )(input, adj, W, a_param, bias, W_res, ln_gamma, ln_beta)
